# Initial kernel scaffold; baseline (speedup 1.0000x reference)
import jax
import jax.numpy as jnp
from jax.experimental import pallas as pl


def kernel(x, emb, W1, b1, W2, b2, W3, b3):
    raise NotImplementedError("write your pallas kernel here")


# speedup vs baseline: 3.0202x; 3.0202x over previous
import functools

import jax
import jax.numpy as jnp
from jax.experimental import pallas as pl
from jax.experimental.pallas import tpu as pltpu
from jax.experimental.pallas import tpu_sc as plsc

_EMBED_DIM = 64
_HIDDEN = 128
_TAGS = 128

_GATHER_WINDOW = 256
_TBL_BLK = 2048


def _sc_gather(table, idx_flat):
    n = idx_flat.shape[0]
    d = table.shape[1]
    idx2 = idx_flat.reshape(1, n)
    mesh = plsc.VectorSubcoreMesh(core_axis_name="core", subcore_axis_name="subcore")

    @functools.partial(
        pl.kernel,
        out_type=jax.ShapeDtypeStruct((n, d), table.dtype),
        mesh=mesh,
    )
    def gather_kernel(tbl_hbm, idx_hbm, out_hbm):
        def body(idx_vmem, out_vmem):
            pltpu.sync_copy(tbl_hbm.at[idx_vmem.at[0]], out_vmem)

        pltpu.emit_pipeline(
            body,
            grid=(n // _GATHER_WINDOW,),
            in_specs=[pl.BlockSpec((1, _GATHER_WINDOW), lambda i: (0, i))],
            out_specs=[pl.BlockSpec((_GATHER_WINDOW, d), lambda i: (i, 0))],
            core_axis_name=("core", "subcore"),
            dimension_semantics=(pltpu.PARALLEL,),
        )(idx_hbm, out_hbm)

    return gather_kernel(table, idx2)


def _mlp_body(e_ref, w1_ref, b1_ref, w2_ref, b2_ref, w3_ref, b3_ref, o_ref):
    h = jnp.dot(e_ref[...], w1_ref[...], preferred_element_type=jnp.float32)
    h = jnp.maximum(h + b1_ref[...], 0.0)
    h = jnp.dot(h, w2_ref[...], preferred_element_type=jnp.float32)
    h = jnp.maximum(h + b2_ref[...], 0.0)
    o_ref[...] = jnp.dot(h, w3_ref[...], preferred_element_type=jnp.float32) + b3_ref[...]


def _tc_table_mlp(emb, W1, b1, W2, b2, W3, b3):
    v = emb.shape[0]
    grid = pl.cdiv(v, _TBL_BLK)
    v_pad = grid * _TBL_BLK
    return pl.pallas_call(
        _mlp_body,
        grid=(grid,),
        in_specs=[
            pl.BlockSpec((_TBL_BLK, _EMBED_DIM), lambda i: (i, 0)),
            pl.BlockSpec((_EMBED_DIM, _HIDDEN), lambda i: (0, 0)),
            pl.BlockSpec((1, _HIDDEN), lambda i: (0, 0)),
            pl.BlockSpec((_HIDDEN, _HIDDEN), lambda i: (0, 0)),
            pl.BlockSpec((1, _HIDDEN), lambda i: (0, 0)),
            pl.BlockSpec((_HIDDEN, _TAGS), lambda i: (0, 0)),
            pl.BlockSpec((1, _TAGS), lambda i: (0, 0)),
        ],
        out_specs=pl.BlockSpec((_TBL_BLK, _TAGS), lambda i: (i, 0)),
        out_shape=jax.ShapeDtypeStruct((v_pad, _TAGS), jnp.float32),
    )(emb, W1, b1.reshape(1, -1), W2, b2.reshape(1, -1), W3, b3.reshape(1, -1))


def kernel(x, emb, W1, b1, W2, b2, W3, b3):
    b, l = x.shape
    idx = x.reshape(-1).astype(jnp.int32)
    out_table = _tc_table_mlp(emb, W1, b1, W2, b2, W3, b3)
    out = _sc_gather(out_table, idx)
    return out.reshape(b, l, _TAGS)


# speedup vs baseline: 3.6290x; 1.2016x over previous
import functools

import jax
import jax.numpy as jnp
from jax.experimental import pallas as pl
from jax.experimental.pallas import tpu as pltpu
from jax.experimental.pallas import tpu_sc as plsc

_EMBED_DIM = 64
_HIDDEN = 128
_TAGS = 128

_ROWS_PER_STEP = 8
_TBL_BLK = 2048


def _sc_gather_out(table, x):
    b, l = x.shape
    d = table.shape[1]
    mesh = plsc.VectorSubcoreMesh(core_axis_name="core", subcore_axis_name="subcore")

    @functools.partial(
        pl.kernel,
        out_type=jax.ShapeDtypeStruct((b, l, d), table.dtype),
        mesh=mesh,
    )
    def gather_kernel(tbl_hbm, idx_hbm, out_hbm):
        def body(idx_vmem, out_vmem):
            for j in range(_ROWS_PER_STEP):
                pltpu.sync_copy(tbl_hbm.at[idx_vmem.at[j]], out_vmem.at[j])

        pltpu.emit_pipeline(
            body,
            grid=(b // _ROWS_PER_STEP,),
            in_specs=[pl.BlockSpec((_ROWS_PER_STEP, l), lambda i: (i, 0))],
            out_specs=[pl.BlockSpec((_ROWS_PER_STEP, l, d), lambda i: (i, 0, 0))],
            core_axis_name=("core", "subcore"),
            dimension_semantics=(pltpu.PARALLEL,),
        )(idx_hbm, out_hbm)

    return gather_kernel(table, x)


def _mlp_body(e_ref, w1_ref, b1_ref, w2_ref, b2_ref, w3_ref, b3_ref, o_ref):
    h = jnp.dot(e_ref[...], w1_ref[...], preferred_element_type=jnp.float32)
    h = jnp.maximum(h + b1_ref[...], 0.0)
    h = jnp.dot(h, w2_ref[...], preferred_element_type=jnp.float32)
    h = jnp.maximum(h + b2_ref[...], 0.0)
    o_ref[...] = jnp.dot(h, w3_ref[...], preferred_element_type=jnp.float32) + b3_ref[...]


def _tc_table_mlp(emb, W1, b1, W2, b2, W3, b3):
    v = emb.shape[0]
    grid = pl.cdiv(v, _TBL_BLK)
    v_pad = grid * _TBL_BLK
    return pl.pallas_call(
        _mlp_body,
        grid=(grid,),
        in_specs=[
            pl.BlockSpec((_TBL_BLK, _EMBED_DIM), lambda i: (i, 0)),
            pl.BlockSpec((_EMBED_DIM, _HIDDEN), lambda i: (0, 0)),
            pl.BlockSpec((1, _HIDDEN), lambda i: (0, 0)),
            pl.BlockSpec((_HIDDEN, _HIDDEN), lambda i: (0, 0)),
            pl.BlockSpec((1, _HIDDEN), lambda i: (0, 0)),
            pl.BlockSpec((_HIDDEN, _TAGS), lambda i: (0, 0)),
            pl.BlockSpec((1, _TAGS), lambda i: (0, 0)),
        ],
        out_specs=pl.BlockSpec((_TBL_BLK, _TAGS), lambda i: (i, 0)),
        out_shape=jax.ShapeDtypeStruct((v_pad, _TAGS), jnp.float32),
    )(emb, W1, b1.reshape(1, -1), W2, b2.reshape(1, -1), W3, b3.reshape(1, -1))


def kernel(x, emb, W1, b1, W2, b2, W3, b3):
    out_table = _tc_table_mlp(emb, W1, b1, W2, b2, W3, b3)
    return _sc_gather_out(out_table, x.astype(jnp.int32))


# speedup vs baseline: 7.5732x; 2.0869x over previous
import functools

import jax
import jax.numpy as jnp
from jax.experimental import pallas as pl
from jax.experimental.pallas import tpu as pltpu
from jax.experimental.pallas import tpu_sc as plsc

_EMBED_DIM = 64
_HIDDEN = 128
_TAGS = 128

_GATHER_WINDOW = 256
_TBL_BLK = 2048


def _sc_gather_out(table, xt3, l):
    nblk, _, w = xt3.shape
    d = table.shape[1]
    mesh = plsc.VectorSubcoreMesh(core_axis_name="core", subcore_axis_name="subcore")
    b = nblk * w // l
    per_l = b // w

    @functools.partial(
        pl.kernel,
        out_type=jax.ShapeDtypeStruct((l, b, d), table.dtype),
        mesh=mesh,
    )
    def gather_kernel(tbl_hbm, idx_hbm, out_hbm):
        def body(idx_vmem, out_vmem):
            pltpu.sync_copy(tbl_hbm.at[idx_vmem.at[0, 0]], out_vmem.at[0])

        pltpu.emit_pipeline(
            body,
            grid=(nblk,),
            in_specs=[pl.BlockSpec((1, 1, w), lambda i: (i, 0, 0))],
            out_specs=[
                pl.BlockSpec((1, w, d), lambda i: (i // per_l, i % per_l, 0))
            ],
            core_axis_name=("core", "subcore"),
            dimension_semantics=(pltpu.PARALLEL,),
        )(idx_hbm, out_hbm)

    return gather_kernel(table, xt3)


def _mlp_body(et_ref, w1_ref, b1_ref, w2_ref, b2_ref, w3_ref, b3_ref, o_ref):
    h = jax.lax.dot_general(
        et_ref[...], w1_ref[...], (((0,), (0,)), ((), ())),
        preferred_element_type=jnp.float32,
    )
    h = jnp.maximum(h + b1_ref[...], 0.0)
    h = jnp.dot(h, w2_ref[...], preferred_element_type=jnp.float32)
    h = jnp.maximum(h + b2_ref[...], 0.0)
    o_ref[...] = jnp.dot(h, w3_ref[...], preferred_element_type=jnp.float32) + b3_ref[...]


def _tc_table_mlp(embt, W1, b1, W2, b2, W3, b3):
    v = embt.shape[1]
    grid = pl.cdiv(v, _TBL_BLK)
    v_pad = grid * _TBL_BLK
    return pl.pallas_call(
        _mlp_body,
        grid=(grid,),
        in_specs=[
            pl.BlockSpec((_EMBED_DIM, _TBL_BLK), lambda i: (0, i)),
            pl.BlockSpec((_EMBED_DIM, _HIDDEN), lambda i: (0, 0)),
            pl.BlockSpec((1, _HIDDEN), lambda i: (0, 0)),
            pl.BlockSpec((_HIDDEN, _HIDDEN), lambda i: (0, 0)),
            pl.BlockSpec((1, _HIDDEN), lambda i: (0, 0)),
            pl.BlockSpec((_HIDDEN, _TAGS), lambda i: (0, 0)),
            pl.BlockSpec((1, _TAGS), lambda i: (0, 0)),
        ],
        out_specs=pl.BlockSpec((_TBL_BLK, _TAGS), lambda i: (i, 0)),
        out_shape=jax.ShapeDtypeStruct((v_pad, _TAGS), jnp.float32),
    )(embt, W1, b1.reshape(1, -1), W2, b2.reshape(1, -1), W3, b3.reshape(1, -1))


def kernel(x, emb, W1, b1, W2, b2, W3, b3):
    b, l = x.shape
    w = _GATHER_WINDOW
    xt3 = x.astype(jnp.int32).T.reshape(l * b // w, 1, w)
    table = _tc_table_mlp(emb.T, W1, b1, W2, b2, W3, b3)
    out_t = _sc_gather_out(table, xt3, l)
    return out_t.transpose(1, 0, 2)


# speedup vs baseline: 8.3984x; 1.1090x over previous
import functools

import jax
import jax.numpy as jnp
from jax.experimental import pallas as pl
from jax.experimental.pallas import tpu as pltpu
from jax.experimental.pallas import tpu_sc as plsc

_EMBED_DIM = 64
_HIDDEN = 128
_TAGS = 128

_GATHER_WINDOW = 256
_TBL_BLK = 4096


def _sc_gather_out(table, xt3, l):
    nblk, _, w = xt3.shape
    d = table.shape[1]
    mesh = plsc.VectorSubcoreMesh(core_axis_name="core", subcore_axis_name="subcore")
    b = nblk * w // l
    per_l = b // w

    @functools.partial(
        pl.kernel,
        out_type=jax.ShapeDtypeStruct((l, b, d), table.dtype),
        mesh=mesh,
    )
    def gather_kernel(tbl_hbm, idx_hbm, out_hbm):
        def body(idx_vmem, out_vmem):
            pltpu.sync_copy(tbl_hbm.at[idx_vmem.at[0, 0]], out_vmem.at[0])

        pltpu.emit_pipeline(
            body,
            grid=(nblk,),
            in_specs=[pl.BlockSpec((1, 1, w), lambda i: (i, 0, 0))],
            out_specs=[
                pl.BlockSpec((1, w, d), lambda i: (i // per_l, i % per_l, 0))
            ],
            core_axis_name=("core", "subcore"),
            dimension_semantics=(pltpu.PARALLEL,),
        )(idx_hbm, out_hbm)

    return gather_kernel(table, xt3)


def _mlp_body(et_ref, w1_ref, b1_ref, w2_ref, b2_ref, w3_ref, b3_ref, o_ref):
    h = jax.lax.dot_general(
        et_ref[...], w1_ref[...], (((0,), (0,)), ((), ())),
        preferred_element_type=jnp.float32,
    )
    h = jnp.maximum(h + b1_ref[...], 0.0)
    h = jnp.dot(h, w2_ref[...], preferred_element_type=jnp.float32)
    h = jnp.maximum(h + b2_ref[...], 0.0)
    o_ref[...] = jnp.dot(h, w3_ref[...], preferred_element_type=jnp.float32) + b3_ref[...]


def _tc_table_mlp(embt, W1, b1, W2, b2, W3, b3):
    v = embt.shape[1]
    grid = pl.cdiv(v, _TBL_BLK)
    v_pad = grid * _TBL_BLK
    return pl.pallas_call(
        _mlp_body,
        grid=(grid,),
        in_specs=[
            pl.BlockSpec((_EMBED_DIM, _TBL_BLK), lambda i: (0, i)),
            pl.BlockSpec((_EMBED_DIM, _HIDDEN), lambda i: (0, 0)),
            pl.BlockSpec((1, _HIDDEN), lambda i: (0, 0)),
            pl.BlockSpec((_HIDDEN, _HIDDEN), lambda i: (0, 0)),
            pl.BlockSpec((1, _HIDDEN), lambda i: (0, 0)),
            pl.BlockSpec((_HIDDEN, _TAGS), lambda i: (0, 0)),
            pl.BlockSpec((1, _TAGS), lambda i: (0, 0)),
        ],
        out_specs=pl.BlockSpec((_TBL_BLK, _TAGS), lambda i: (i, 0)),
        out_shape=jax.ShapeDtypeStruct((v_pad, _TAGS), jnp.float32),
    )(embt, W1, b1.reshape(1, -1), W2, b2.reshape(1, -1), W3, b3.reshape(1, -1))


def kernel(x, emb, W1, b1, W2, b2, W3, b3):
    b, l = x.shape
    w = _GATHER_WINDOW
    xt3 = x.astype(jnp.int32).T.reshape(l * b // w, 1, w)
    table = _tc_table_mlp(emb.T, W1, b1, W2, b2, W3, b3)
    out_t = _sc_gather_out(table, xt3, l)
    return out_t.transpose(1, 0, 2)


# speedup vs baseline: 8.8161x; 1.0497x over previous
import functools

import jax
import jax.numpy as jnp
from jax.experimental import pallas as pl
from jax.experimental.pallas import tpu as pltpu
from jax.experimental.pallas import tpu_sc as plsc

_EMBED_DIM = 64
_HIDDEN = 128
_TAGS = 128

_GATHER_WINDOW = 256
_TBL_BLK = 8192


def _sc_gather_out(table, xt3, l):
    nblk, _, w = xt3.shape
    d = table.shape[1]
    mesh = plsc.VectorSubcoreMesh(core_axis_name="core", subcore_axis_name="subcore")
    b = nblk * w // l
    per_l = b // w

    @functools.partial(
        pl.kernel,
        out_type=jax.ShapeDtypeStruct((l, b, d), table.dtype),
        mesh=mesh,
    )
    def gather_kernel(tbl_hbm, idx_hbm, out_hbm):
        def body(idx_vmem, out_vmem):
            pltpu.sync_copy(tbl_hbm.at[idx_vmem.at[0, 0]], out_vmem.at[0])

        pltpu.emit_pipeline(
            body,
            grid=(nblk,),
            in_specs=[pl.BlockSpec((1, 1, w), lambda i: (i, 0, 0))],
            out_specs=[
                pl.BlockSpec((1, w, d), lambda i: (i // per_l, i % per_l, 0))
            ],
            core_axis_name=("core", "subcore"),
            dimension_semantics=(pltpu.PARALLEL,),
        )(idx_hbm, out_hbm)

    return gather_kernel(table, xt3)


def _mlp_body(et_ref, w1_ref, b1_ref, w2_ref, b2_ref, w3_ref, b3_ref, o_ref):
    h = jax.lax.dot_general(
        et_ref[...], w1_ref[...], (((0,), (0,)), ((), ())),
        preferred_element_type=jnp.float32,
    )
    h = jnp.maximum(h + b1_ref[...], 0.0)
    h = jnp.dot(h, w2_ref[...], preferred_element_type=jnp.float32)
    h = jnp.maximum(h + b2_ref[...], 0.0)
    o_ref[...] = jnp.dot(h, w3_ref[...], preferred_element_type=jnp.float32) + b3_ref[...]


def _tc_table_mlp(embt, W1, b1, W2, b2, W3, b3):
    v = embt.shape[1]
    grid = pl.cdiv(v, _TBL_BLK)
    v_pad = grid * _TBL_BLK
    return pl.pallas_call(
        _mlp_body,
        grid=(grid,),
        in_specs=[
            pl.BlockSpec((_EMBED_DIM, _TBL_BLK), lambda i: (0, i)),
            pl.BlockSpec((_EMBED_DIM, _HIDDEN), lambda i: (0, 0)),
            pl.BlockSpec((1, _HIDDEN), lambda i: (0, 0)),
            pl.BlockSpec((_HIDDEN, _HIDDEN), lambda i: (0, 0)),
            pl.BlockSpec((1, _HIDDEN), lambda i: (0, 0)),
            pl.BlockSpec((_HIDDEN, _TAGS), lambda i: (0, 0)),
            pl.BlockSpec((1, _TAGS), lambda i: (0, 0)),
        ],
        out_specs=pl.BlockSpec((_TBL_BLK, _TAGS), lambda i: (i, 0)),
        out_shape=jax.ShapeDtypeStruct((v_pad, _TAGS), jnp.float32),
    )(embt, W1, b1.reshape(1, -1), W2, b2.reshape(1, -1), W3, b3.reshape(1, -1))


def kernel(x, emb, W1, b1, W2, b2, W3, b3):
    b, l = x.shape
    w = _GATHER_WINDOW
    xt3 = x.astype(jnp.int32).T.reshape(l * b // w, 1, w)
    table = _tc_table_mlp(emb.T, W1, b1, W2, b2, W3, b3)
    out_t = _sc_gather_out(table, xt3, l)
    return out_t.transpose(1, 0, 2)


# speedup vs baseline: 9.0030x; 1.0212x over previous
import functools

import jax
import jax.numpy as jnp
from jax.experimental import pallas as pl
from jax.experimental.pallas import tpu as pltpu
from jax.experimental.pallas import tpu_sc as plsc

_EMBED_DIM = 64
_HIDDEN = 128
_TAGS = 128

_GATHER_WINDOW = 256
_TBL_BLK = 8192


def _sc_gather_out(table, xt3, l):
    nblk, _, w = xt3.shape
    d = table.shape[1]
    mesh = plsc.VectorSubcoreMesh(core_axis_name="core", subcore_axis_name="subcore")
    b = nblk * w // l
    per_l = b // w

    @functools.partial(
        pl.kernel,
        out_type=jax.ShapeDtypeStruct((l, b, d), table.dtype),
        mesh=mesh,
    )
    def gather_kernel(tbl_hbm, idx_hbm, out_hbm):
        def body(idx_vmem, out_vmem):
            pltpu.sync_copy(tbl_hbm.at[idx_vmem.at[0, 0]], out_vmem.at[0])

        pltpu.emit_pipeline(
            body,
            grid=(nblk,),
            in_specs=[pl.BlockSpec((1, 1, w), lambda i: (i, 0, 0))],
            out_specs=[
                pl.BlockSpec((1, w, d), lambda i: (i // per_l, i % per_l, 0))
            ],
            core_axis_name=("core", "subcore"),
            dimension_semantics=(pltpu.PARALLEL,),
        )(idx_hbm, out_hbm)

    return gather_kernel(table, xt3)


def _mlp_body(et_ref, w1_ref, b1_ref, w2_ref, b2_ref, w3_ref, b3_ref, o_ref):
    h = jax.lax.dot_general(
        et_ref[...].astype(jnp.bfloat16), w1_ref[...].astype(jnp.bfloat16),
        (((0,), (0,)), ((), ())),
        preferred_element_type=jnp.float32,
    )
    h = jnp.maximum(h + b1_ref[...], 0.0)
    h = jnp.dot(h.astype(jnp.bfloat16), w2_ref[...].astype(jnp.bfloat16),
                preferred_element_type=jnp.float32)
    h = jnp.maximum(h + b2_ref[...], 0.0)
    o_ref[...] = jnp.dot(h.astype(jnp.bfloat16), w3_ref[...].astype(jnp.bfloat16),
                         preferred_element_type=jnp.float32) + b3_ref[...]


def _tc_table_mlp(embt, W1, b1, W2, b2, W3, b3):
    v = embt.shape[1]
    grid = pl.cdiv(v, _TBL_BLK)
    v_pad = grid * _TBL_BLK
    return pl.pallas_call(
        _mlp_body,
        grid=(grid,),
        in_specs=[
            pl.BlockSpec((_EMBED_DIM, _TBL_BLK), lambda i: (0, i)),
            pl.BlockSpec((_EMBED_DIM, _HIDDEN), lambda i: (0, 0)),
            pl.BlockSpec((1, _HIDDEN), lambda i: (0, 0)),
            pl.BlockSpec((_HIDDEN, _HIDDEN), lambda i: (0, 0)),
            pl.BlockSpec((1, _HIDDEN), lambda i: (0, 0)),
            pl.BlockSpec((_HIDDEN, _TAGS), lambda i: (0, 0)),
            pl.BlockSpec((1, _TAGS), lambda i: (0, 0)),
        ],
        out_specs=pl.BlockSpec((_TBL_BLK, _TAGS), lambda i: (i, 0)),
        out_shape=jax.ShapeDtypeStruct((v_pad, _TAGS), jnp.float32),
    )(embt, W1, b1.reshape(1, -1), W2, b2.reshape(1, -1), W3, b3.reshape(1, -1))


def kernel(x, emb, W1, b1, W2, b2, W3, b3):
    b, l = x.shape
    w = _GATHER_WINDOW
    xt3 = x.astype(jnp.int32).T.reshape(l * b // w, 1, w)
    table = _tc_table_mlp(emb.T, W1, b1, W2, b2, W3, b3)
    out_t = _sc_gather_out(table, xt3, l)
    return out_t.transpose(1, 0, 2)
